# baseline (speedup 1.0000x reference)
import jax
import jax.numpy as jnp
from jax.experimental import pallas as pl
from jax.experimental.pallas import tpu as pltpu
from jax.experimental.pallas import tpu_sc as plsc

_CODEBOOK = 1024
_DIM = 64
_COMMIT = 0.25
_GATHER_WINDOW = 128
_B = 32
_T = 1024
_CHUNK = 8
_NCHUNK = 4


def _idx_body(x_ref, emb_ref, x2_ref, e2_ref, idx_ref):
    xb = x_ref[0]
    emb = emb_ref[...]
    x2 = x2_ref[0]
    e2 = e2_ref[...]
    xe2 = jax.lax.dot_general(
        emb, xb, (((1,), (0,)), ((), ())),
        preferred_element_type=jnp.float32)
    d = (x2 + e2) - xe2
    dmin = jnp.min(d, axis=0)
    iota = jax.lax.broadcasted_iota(jnp.int32, d.shape, 0)
    idx_ref[0, :] = jnp.min(jnp.where(d == dmin[None, :], iota, _CODEBOOK),
                            axis=0)


def _idx_chunk(x3, emb2, x2, e2, off):
    return pl.pallas_call(
        _idx_body,
        grid=(_CHUNK,),
        in_specs=[
            pl.BlockSpec((1, _DIM, _T), lambda b: (b + off, 0, 0)),
            pl.BlockSpec((_CODEBOOK, _DIM), lambda b: (0, 0)),
            pl.BlockSpec((1, 1, _T), lambda b: (b + off, 0, 0)),
            pl.BlockSpec((_CODEBOOK, 1), lambda b: (0, 0)),
        ],
        out_specs=pl.BlockSpec((1, _T), lambda b: (0, b)),
        out_shape=jax.ShapeDtypeStruct((1, _CHUNK * _T), jnp.int32),
    )(x3, emb2, x2, e2)


def _sc_gather(emb_pad, idx_flat, n_rows):
    mesh = plsc.VectorSubcoreMesh(core_axis_name="c", subcore_axis_name="s")

    @pl.kernel(out_type=jax.ShapeDtypeStruct((n_rows, _DIM), jnp.float32),
               mesh=mesh,
               scratch_types=[pltpu.VMEM((_GATHER_WINDOW, 128), jnp.float32)])
    def gather_kernel(emb_hbm, i_hbm, o_hbm, tmp_ref):
        def body(i_vmem, o_vmem):
            pltpu.sync_copy(emb_hbm.at[i_vmem.at[0]], tmp_ref)
            o_vmem[...] = tmp_ref[:, :_DIM]

        pltpu.emit_pipeline(
            body,
            grid=(n_rows // _GATHER_WINDOW,),
            in_specs=[pl.BlockSpec((1, _GATHER_WINDOW),
                                   index_map=lambda i: (0, i))],
            out_specs=[pl.BlockSpec((_GATHER_WINDOW, _DIM),
                                    index_map=lambda i: (i, 0))],
            core_axis_name=("c", "s"),
            dimension_semantics=(pltpu.PARALLEL,),
        )(i_hbm, o_hbm)

    return gather_kernel(emb_pad, idx_flat)


def _st_body(x_ref, q_ref, out_ref, loss_ref):
    b = pl.program_id(0)
    xb = x_ref[0]
    qT = jnp.transpose(q_ref[0])
    out_ref[0] = xb + (qT - xb)
    part = jnp.sum((qT - xb) ** 2)

    @pl.when(b == 0)
    def _():
        loss_ref[0, 0] = 0.0

    loss_ref[0, 0] += part


def _st_body_alias(x_ref, q_ref, prev_ref, out_ref, loss_ref):
    _st_body(x_ref, q_ref, out_ref, loss_ref)


def _st_chunk_first(x3, q_chunk, off):
    return pl.pallas_call(
        _st_body,
        grid=(_CHUNK,),
        in_specs=[
            pl.BlockSpec((1, _DIM, _T), lambda b: (b + off, 0, 0)),
            pl.BlockSpec((1, _T, _DIM), lambda b: (b, 0, 0)),
        ],
        out_specs=[
            pl.BlockSpec((1, _DIM, _T), lambda b: (b + off, 0, 0)),
            pl.BlockSpec(block_shape=(1, 1), index_map=lambda b: (0, 0),
                         memory_space=pltpu.MemorySpace.SMEM),
        ],
        out_shape=[
            jax.ShapeDtypeStruct((_B, _DIM, _T), jnp.float32),
            jax.ShapeDtypeStruct((1, 1), jnp.float32),
        ],
    )(x3, q_chunk)


def _st_chunk_alias(x3, q_chunk, prev, off):
    return pl.pallas_call(
        _st_body_alias,
        grid=(_CHUNK,),
        in_specs=[
            pl.BlockSpec((1, _DIM, _T), lambda b: (b + off, 0, 0)),
            pl.BlockSpec((1, _T, _DIM), lambda b: (b, 0, 0)),
            pl.BlockSpec(memory_space=pltpu.MemorySpace.HBM),
        ],
        out_specs=[
            pl.BlockSpec((1, _DIM, _T), lambda b: (b + off, 0, 0)),
            pl.BlockSpec(block_shape=(1, 1), index_map=lambda b: (0, 0),
                         memory_space=pltpu.MemorySpace.SMEM),
        ],
        out_shape=[
            jax.ShapeDtypeStruct((_B, _DIM, _T), jnp.float32),
            jax.ShapeDtypeStruct((1, 1), jnp.float32),
        ],
        input_output_aliases={2: 0},
    )(x3, q_chunk, prev)


def kernel(x, emb_weight):
    B, C, H, W = x.shape
    T = H * W
    x3 = x.reshape(B, C, T)
    flat_x = jnp.transpose(x, (0, 2, 3, 1)).reshape(-1, C)
    x2 = jnp.sum(flat_x ** 2, axis=1).reshape(B, 1, T)
    e2 = jnp.sum(emb_weight ** 2, axis=1).reshape(_CODEBOOK, 1)
    emb_x2 = emb_weight * 2.0
    emb_pad = jnp.concatenate(
        [emb_weight, jnp.zeros((_CODEBOOK, 128 - _DIM), jnp.float32)], axis=1)

    idxs = [_idx_chunk(x3, emb_x2, x2, e2, k * _CHUNK)
            for k in range(_NCHUNK)]
    qs = [_sc_gather(emb_pad, i, _CHUNK * T) for i in idxs]

    out = None
    losses = []
    for k in range(_NCHUNK):
        qk = qs[k].reshape(_CHUNK, T, _DIM)
        if out is None:
            out, lk = _st_chunk_first(x3, qk, 0)
        else:
            out, lk = _st_chunk_alias(x3, qk, out, k * _CHUNK)
        losses.append(lk[0, 0])
    q3 = out
    loss_sum = losses[0]
    for lk in losses[1:]:
        loss_sum = loss_sum + lk

    m = loss_sum / (B * C * H * W)
    loss = m + _COMMIT * m
    return q3.reshape(B, C, H, W), loss


# speedup vs baseline: 1.0598x; 1.0598x over previous
import jax
import jax.numpy as jnp
from jax.experimental import pallas as pl
from jax.experimental.pallas import tpu as pltpu
from jax.experimental.pallas import tpu_sc as plsc

_CODEBOOK = 1024
_DIM = 64
_COMMIT = 0.25
_GATHER_WINDOW = 128
_B = 32
_T = 1024
_CHUNK = 16


def _idx_body(x_ref, emb_ref, x2_ref, e2_ref, idx_ref, loss_ref):
    xb = x_ref[0]
    emb = emb_ref[...]
    x2 = x2_ref[0]
    e2 = e2_ref[...]
    xe2 = jax.lax.dot_general(
        emb, xb, (((1,), (0,)), ((), ())),
        preferred_element_type=jnp.float32)
    d = (x2 + e2) - xe2
    dmin = jnp.min(d, axis=0)
    iota = jax.lax.broadcasted_iota(jnp.int32, d.shape, 0)
    idx_ref[0, :] = jnp.min(jnp.where(d == dmin[None, :], iota, _CODEBOOK),
                            axis=0)
    part = jnp.sum(dmin)

    @pl.when(pl.program_id(0) == 0)
    def _():
        loss_ref[0, 0] = 0.0

    loss_ref[0, 0] += part


def _idx_chunk(x3, emb2, x2, e2, off):
    return pl.pallas_call(
        _idx_body,
        grid=(_CHUNK,),
        in_specs=[
            pl.BlockSpec((1, _DIM, _T), lambda b: (b + off, 0, 0)),
            pl.BlockSpec((_CODEBOOK, _DIM), lambda b: (0, 0)),
            pl.BlockSpec((1, 1, _T), lambda b: (b + off, 0, 0)),
            pl.BlockSpec((_CODEBOOK, 1), lambda b: (0, 0)),
        ],
        out_specs=[
            pl.BlockSpec((1, _T), lambda b: (0, b)),
            pl.BlockSpec(block_shape=(1, 1), index_map=lambda b: (0, 0),
                         memory_space=pltpu.MemorySpace.SMEM),
        ],
        out_shape=[
            jax.ShapeDtypeStruct((1, _CHUNK * _T), jnp.int32),
            jax.ShapeDtypeStruct((1, 1), jnp.float32),
        ],
    )(x3, emb2, x2, e2)


def _sc_gather(emb_pad, idx_flat, n_rows):
    mesh = plsc.VectorSubcoreMesh(core_axis_name="c", subcore_axis_name="s")

    @pl.kernel(out_type=jax.ShapeDtypeStruct((n_rows, _DIM), jnp.float32),
               mesh=mesh,
               scratch_types=[pltpu.VMEM((_GATHER_WINDOW, 128), jnp.float32)])
    def gather_kernel(emb_hbm, i_hbm, o_hbm, tmp_ref):
        def body(i_vmem, o_vmem):
            pltpu.sync_copy(emb_hbm.at[i_vmem.at[0]], tmp_ref)
            o_vmem[...] = tmp_ref[:, :_DIM]

        pltpu.emit_pipeline(
            body,
            grid=(n_rows // _GATHER_WINDOW,),
            in_specs=[pl.BlockSpec((1, _GATHER_WINDOW),
                                   index_map=lambda i: (0, i))],
            out_specs=[pl.BlockSpec((_GATHER_WINDOW, _DIM),
                                    index_map=lambda i: (i, 0))],
            core_axis_name=("c", "s"),
            dimension_semantics=(pltpu.PARALLEL,),
        )(i_hbm, o_hbm)

    return gather_kernel(emb_pad, idx_flat)


def _tr_body(q_ref, out_ref):
    out_ref[0] = jnp.transpose(q_ref[0])


def _tr_body_alias(q_ref, prev_ref, out_ref):
    _tr_body(q_ref, out_ref)


def _tr_chunk_first(q_chunk, off):
    return pl.pallas_call(
        _tr_body,
        grid=(_CHUNK,),
        in_specs=[
            pl.BlockSpec((1, _T, _DIM), lambda b: (b, 0, 0)),
        ],
        out_specs=pl.BlockSpec((1, _DIM, _T), lambda b: (b + off, 0, 0)),
        out_shape=jax.ShapeDtypeStruct((_B, _DIM, _T), jnp.float32),
    )(q_chunk)


def _tr_chunk_alias(q_chunk, prev, off):
    return pl.pallas_call(
        _tr_body_alias,
        grid=(_CHUNK,),
        in_specs=[
            pl.BlockSpec((1, _T, _DIM), lambda b: (b, 0, 0)),
            pl.BlockSpec(memory_space=pltpu.MemorySpace.HBM),
        ],
        out_specs=pl.BlockSpec((1, _DIM, _T), lambda b: (b + off, 0, 0)),
        out_shape=jax.ShapeDtypeStruct((_B, _DIM, _T), jnp.float32),
        input_output_aliases={1: 0},
    )(q_chunk, prev)


def kernel(x, emb_weight):
    B, C, H, W = x.shape
    T = H * W
    x3 = x.reshape(B, C, T)
    flat_x = jnp.transpose(x, (0, 2, 3, 1)).reshape(-1, C)
    x2 = jnp.sum(flat_x ** 2, axis=1).reshape(B, 1, T)
    e2 = jnp.sum(emb_weight ** 2, axis=1).reshape(_CODEBOOK, 1)
    emb_x2 = emb_weight * 2.0
    emb_pad = jnp.concatenate(
        [emb_weight, jnp.zeros((_CODEBOOK, 128 - _DIM), jnp.float32)], axis=1)

    idx_a, loss_a = _idx_chunk(x3, emb_x2, x2, e2, 0)
    idx_b, loss_b = _idx_chunk(x3, emb_x2, x2, e2, _CHUNK)
    q_a = _sc_gather(emb_pad, idx_a, _CHUNK * T)
    q_b = _sc_gather(emb_pad, idx_b, _CHUNK * T)

    out_a = _tr_chunk_first(q_a.reshape(_CHUNK, T, _DIM), 0)
    q3 = _tr_chunk_alias(q_b.reshape(_CHUNK, T, _DIM), out_a, _CHUNK)

    m = (loss_a[0, 0] + loss_b[0, 0]) / (B * C * H * W)
    loss = m + _COMMIT * m
    return q3.reshape(B, C, H, W), loss
